# baseline (speedup 1.0000x reference)
import functools

import jax
import jax.numpy as jnp
from jax import lax
from jax.experimental import pallas as pl
from jax.experimental.pallas import tpu as pltpu
from jax.experimental.pallas import tpu_sc as plsc

NU = 50000
NI = 50000
NE = 200000
D = 128

BLK = 128
TPB = 98
EPT = TPB * BLK
EPAD = 16 * EPT
STG = 2048
NSTG = 7
CHUNK = 13568
ACCR = CHUNK + BLK
NBUCKET = 4
ROUNDS = 2
OUTPAD = NBUCKET * CHUNK

_ZSLICES = ((0, 128), (128, 128), (256, 128), (384, 128),
            (512, 128), (640, 128), (768, 88))
_WSLICES = ((0, 128), (128, 128), (256, 128), (384, 128),
            (512, 128), (640, 128), (768, 80))

_MESH = plsc.VectorSubcoreMesh(core_axis_name="c", subcore_axis_name="s")


def _stage_sizes():
    return [(st * STG, STG if st < 6 else EPT - 6 * STG) for st in range(NSTG)]


def _segsum_body(table, srcp, dstp, zeros_h, out,
                 acc, sidx_c, didx_c, lblk, sblk, rows, sem):
    c = lax.axis_index("c")
    s = lax.axis_index("s")
    for r in range(ROUNDS):
        lo = (2 * r + c) * CHUNK
        base = s * (ACCR // 16)
        for qo, qn in _ZSLICES:
            pltpu.sync_copy(zeros_h.at[pl.ds(0, qn)],
                            acc.at[pl.ds(base + qo, qn)])
        plsc.subcore_barrier()
        for soff, ssz in _stage_sizes():
            pltpu.sync_copy(srcp.at[pl.ds(s * EPT + soff, ssz)],
                            sidx_c.at[pl.ds(0, ssz)])
            pltpu.sync_copy(dstp.at[pl.ds(s * EPT + soff, ssz)],
                            didx_c.at[pl.ds(0, ssz)])

            def eblock(i, carry):
                for k in range(BLK // 16):
                    d = didx_c[pl.ds(i * BLK + k * 16, 16)]
                    inr = (d >= lo) & (d < lo + CHUNK)
                    lblk[pl.ds(k * 16, 16)] = jnp.where(inr, d - lo, CHUNK)
                    sblk[pl.ds(k * 16, 16)] = sidx_c[pl.ds(i * BLK + k * 16,
                                                           16)]
                pltpu.async_copy(table.at[sblk], rows, sem).wait()
                pltpu.sync_copy(rows, acc.at[lblk], add=True)
                return carry

            lax.fori_loop(0, ssz // BLK, eblock, 0)
        plsc.subcore_barrier()
        wbase = s * (CHUNK // 16)
        for qo, qn in _WSLICES:
            pltpu.sync_copy(acc.at[pl.ds(wbase + qo, qn)],
                            out.at[pl.ds(lo + wbase + qo, qn)])
        plsc.subcore_barrier()


@functools.partial(
    pl.kernel,
    out_type=jax.ShapeDtypeStruct((OUTPAD, D), jnp.float32),
    mesh=_MESH,
    scratch_types=[
        pltpu.VMEM_SHARED((ACCR, D), jnp.float32),
        pltpu.VMEM((STG,), jnp.int32),
        pltpu.VMEM((STG,), jnp.int32),
        pltpu.VMEM((BLK,), jnp.int32),
        pltpu.VMEM((BLK,), jnp.int32),
        pltpu.VMEM((BLK, D), jnp.float32),
        pltpu.SemaphoreType.DMA,
    ],
)
def _segsum(table, srcp, dstp, zeros_h, out,
            acc, sidx_c, didx_c, lblk, sblk, rows, sem):
    _segsum_body(table, srcp, dstp, zeros_h, out,
                 acc, sidx_c, didx_c, lblk, sblk, rows, sem)


def _counts_body(dsts, ones_h, zeros_h, outs, cnt, didx_c, lblk, onesv, sem):
    c = lax.axis_index("c")
    s = lax.axis_index("s")
    pltpu.sync_copy(ones_h, onesv)
    for e in range(3):
        for r in range(ROUNDS):
            lo = (2 * r + c) * CHUNK
            base = s * (ACCR // 16)
            for qo, qn in _ZSLICES:
                pltpu.sync_copy(zeros_h.at[pl.ds(0, qn)],
                                cnt.at[pl.ds(base + qo, qn)])
            plsc.subcore_barrier()
            for soff, ssz in _stage_sizes():
                pltpu.sync_copy(dsts[e].at[pl.ds(s * EPT + soff, ssz)],
                                didx_c.at[pl.ds(0, ssz)])

                def eblock(i, carry):
                    for k in range(BLK // 16):
                        d = didx_c[pl.ds(i * BLK + k * 16, 16)]
                        inr = (d >= lo) & (d < lo + CHUNK)
                        lblk[pl.ds(k * 16, 16)] = jnp.where(inr, d - lo,
                                                            CHUNK)
                    pltpu.sync_copy(onesv, cnt.at[lblk], add=True)
                    return carry

                lax.fori_loop(0, ssz // BLK, eblock, 0)
            plsc.subcore_barrier()
            wbase = s * (CHUNK // 16)
            for qo, qn in _WSLICES:
                pltpu.sync_copy(cnt.at[pl.ds(wbase + qo, qn)],
                                outs[e].at[pl.ds(lo + wbase + qo, qn)])
            plsc.subcore_barrier()


@functools.partial(
    pl.kernel,
    out_type=[jax.ShapeDtypeStruct((OUTPAD, D), jnp.float32)] * 3,
    mesh=_MESH,
    scratch_types=[
        pltpu.VMEM_SHARED((ACCR, D), jnp.float32),
        pltpu.VMEM((STG,), jnp.int32),
        pltpu.VMEM((BLK,), jnp.int32),
        pltpu.VMEM((BLK, D), jnp.float32),
        pltpu.SemaphoreType.DMA,
    ],
)
def _counts(dst_f, dst_c, dst_cb, ones_h, zeros_h, out_f, out_c, out_cb,
            cnt, didx_c, lblk, onesv, sem):
    _counts_body((dst_f, dst_c, dst_cb), ones_h, zeros_h,
                 (out_f, out_c, out_cb), cnt, didx_c, lblk, onesv, sem)


RBLK = 400


def _combine_u_kernel(sf, cf, scb, ccb, fu, wf, wcb, w0, bf, bcb, b0, out,
                      *, leaky):
    cfv = cf[...]
    ccbv = ccb[...]
    xf = sf[...] / jnp.maximum(cfv, 1.0)
    xcb = scb[...] / jnp.maximum(ccbv, 1.0)
    acc = jnp.dot(xf, wf[...], preferred_element_type=jnp.float32)
    acc = acc + jnp.dot(xcb, wcb[...], preferred_element_type=jnp.float32)
    acc = acc + jnp.dot(fu[...], w0[...], preferred_element_type=jnp.float32)
    bias = (b0[...] + jnp.where(cfv > 0.5, bf[...], 0.0)
            + jnp.where(ccbv > 0.5, bcb[...], 0.0))
    y = acc + bias
    if leaky:
        y = jnp.where(y >= 0, y, 0.01 * y)
    out[...] = y


def _combine_i_kernel(sc_, cc, fi, wc, w0, bc, b0, out, *, leaky):
    ccv = cc[...]
    xc = sc_[...] / jnp.maximum(ccv, 1.0)
    acc = jnp.dot(xc, wc[...], preferred_element_type=jnp.float32)
    acc = acc + jnp.dot(fi[...], w0[...], preferred_element_type=jnp.float32)
    bias = b0[...] + jnp.where(ccv > 0.5, bc[...], 0.0)
    y = acc + bias
    if leaky:
        y = jnp.where(y >= 0, y, 0.01 * y)
    out[...] = y


def _row_spec():
    return pl.BlockSpec((RBLK, D), lambda i: (i, 0))


def _cnt_spec():
    return pl.BlockSpec((RBLK, 1), lambda i: (i, 0))


def _w_spec():
    return pl.BlockSpec((D, D), lambda i: (0, 0))


def _b_spec():
    return pl.BlockSpec((1, D), lambda i: (0, 0))


def _combine_u(sf, cf, scb, ccb, fu, wf, wcb, w0, bf, bcb, b0, leaky):
    return pl.pallas_call(
        functools.partial(_combine_u_kernel, leaky=leaky),
        grid=(NU // RBLK,),
        in_specs=[_row_spec(), _cnt_spec(), _row_spec(), _cnt_spec(),
                  _row_spec(), _w_spec(), _w_spec(), _w_spec(),
                  _b_spec(), _b_spec(), _b_spec()],
        out_specs=_row_spec(),
        out_shape=jax.ShapeDtypeStruct((NU, D), jnp.float32),
    )(sf, cf, scb, ccb, fu, wf, wcb, w0,
      bf.reshape(1, D), bcb.reshape(1, D), b0.reshape(1, D))


def _combine_i(sc_, cc, fi, wc, w0, bc, b0, leaky):
    return pl.pallas_call(
        functools.partial(_combine_i_kernel, leaky=leaky),
        grid=(NI // RBLK,),
        in_specs=[_row_spec(), _cnt_spec(), _row_spec(),
                  _w_spec(), _w_spec(), _b_spec(), _b_spec()],
        out_specs=_row_spec(),
        out_shape=jax.ShapeDtypeStruct((NI, D), jnp.float32),
    )(sc_, cc, fi, wc, w0, bc.reshape(1, D), b0.reshape(1, D))


def _pad_edges(e):
    src = jnp.concatenate([e[0], jnp.zeros((EPAD - NE,), jnp.int32)])
    dst = jnp.concatenate([e[1], jnp.full((EPAD - NE,), NU, jnp.int32)])
    return src, dst


def kernel(e_follows, e_clicks, e_clickedby, emb_user, emb_item,
           l1_w0, l1_b0, l1_wf, l1_bf, l1_wc, l1_bc, l1_wcb, l1_bcb,
           l2_w0, l2_b0, l2_wf, l2_bf, l2_wc, l2_bc, l2_wcb, l2_bcb):
    src_f, dst_f = _pad_edges(e_follows)
    src_c, dst_c = _pad_edges(e_clicks)
    src_cb, dst_cb = _pad_edges(e_clickedby)
    zeros_h = jnp.zeros((BLK, D), jnp.float32)
    ones_hD = jnp.ones((BLK, D), jnp.float32)

    cnt_f, cnt_c, cnt_cb = _counts(dst_f, dst_c, dst_cb, ones_hD, zeros_h)
    cf = cnt_f[:NU, 0:1]
    cc = cnt_c[:NI, 0:1]
    ccb = cnt_cb[:NU, 0:1]

    def seg(tab, src, dst):
        return _segsum(tab, src, dst, zeros_h)[:NU]

    def layer(fu, fi, w0, b0, wf, bf, wc, bc, wcb, bcb, leaky):
        sf = seg(fu, src_f, dst_f)
        scb = seg(fi, src_cb, dst_cb)
        sc_ = seg(fu, src_c, dst_c)
        hu = _combine_u(sf, cf, scb, ccb, fu, wf, wcb, w0, bf, bcb, b0,
                        leaky)
        hi = _combine_i(sc_, cc, fi, wc, w0, bc, b0, leaky)
        return hu, hi

    hu, hi = layer(emb_user, emb_item, l1_w0, l1_b0, l1_wf, l1_bf,
                   l1_wc, l1_bc, l1_wcb, l1_bcb, True)
    hu, hi = layer(hu, hi, l2_w0, l2_b0, l2_wf, l2_bf,
                   l2_wc, l2_bc, l2_wcb, l2_bcb, False)
    return hu, hi


# speedup vs baseline: 1.0145x; 1.0145x over previous
import functools

import jax
import jax.numpy as jnp
from jax import lax
from jax.experimental import pallas as pl
from jax.experimental.pallas import tpu as pltpu
from jax.experimental.pallas import tpu_sc as plsc

NU = 50000
NI = 50000
NE = 200000
D = 128

BLK = 64
EPT = 12544
EPAD = 16 * EPT
STG = 2048
NSTG = 7
CHUNK = 13568
ACCR = CHUNK + BLK
NBUCKET = 4
ROUNDS = 2
OUTPAD = NBUCKET * CHUNK

_ZSLICES = ((0, 128), (128, 128), (256, 128), (384, 128),
            (512, 128), (640, 128), (768, 88))
_WSLICES = ((0, 128), (128, 128), (256, 128), (384, 128),
            (512, 128), (640, 128), (768, 80))

_MESH = plsc.VectorSubcoreMesh(core_axis_name="c", subcore_axis_name="s")


def _stage_sizes():
    return [(st * STG, STG if st < 6 else EPT - 6 * STG) for st in range(NSTG)]


def _segsum_body(table, srcp, dstp, zeros_h, out,
                 acc, sidx_c, didx_c, lblk, sblk, rows, lblk2, sblk2, rows2,
                 sem, sem2, sem3, sem4):
    c = lax.axis_index("c")
    s = lax.axis_index("s")
    for r in range(ROUNDS):
        lo = (2 * r + c) * CHUNK
        base = s * (ACCR // 16)
        for qo, qn in _ZSLICES:
            pltpu.sync_copy(zeros_h.at[pl.ds(0, qn)],
                            acc.at[pl.ds(base + qo, qn)])
        plsc.subcore_barrier()
        for soff, ssz in _stage_sizes():
            pltpu.sync_copy(srcp.at[pl.ds(s * EPT + soff, ssz)],
                            sidx_c.at[pl.ds(0, ssz)])
            pltpu.sync_copy(dstp.at[pl.ds(s * EPT + soff, ssz)],
                            didx_c.at[pl.ds(0, ssz)])

            def epair(i, carry):
                o0 = 2 * i * BLK
                for k in range(BLK // 16):
                    d = didx_c[pl.ds(o0 + k * 16, 16)]
                    inr = (d >= lo) & (d < lo + CHUNK)
                    lblk[pl.ds(k * 16, 16)] = jnp.where(inr, d - lo, CHUNK)
                    sblk[pl.ds(k * 16, 16)] = sidx_c[pl.ds(o0 + k * 16, 16)]
                d0 = pltpu.async_copy(table.at[sblk], rows, sem)
                o1 = o0 + BLK
                for k in range(BLK // 16):
                    d = didx_c[pl.ds(o1 + k * 16, 16)]
                    inr = (d >= lo) & (d < lo + CHUNK)
                    lblk2[pl.ds(k * 16, 16)] = jnp.where(inr, d - lo, CHUNK)
                    sblk2[pl.ds(k * 16, 16)] = sidx_c[pl.ds(o1 + k * 16, 16)]
                d1 = pltpu.async_copy(table.at[sblk2], rows2, sem2)
                d0.wait()
                s0 = pltpu.async_copy(rows, acc.at[lblk], sem3, add=True)
                d1.wait()
                s1 = pltpu.async_copy(rows2, acc.at[lblk2], sem4, add=True)
                s0.wait()
                s1.wait()
                return carry

            lax.fori_loop(0, ssz // (2 * BLK), epair, 0)
        plsc.subcore_barrier()
        wbase = s * (CHUNK // 16)
        for qo, qn in _WSLICES:
            pltpu.sync_copy(acc.at[pl.ds(wbase + qo, qn)],
                            out.at[pl.ds(lo + wbase + qo, qn)])
        plsc.subcore_barrier()


@functools.partial(
    pl.kernel,
    out_type=jax.ShapeDtypeStruct((OUTPAD, D), jnp.float32),
    mesh=_MESH,
    scratch_types=[
        pltpu.VMEM_SHARED((ACCR, D), jnp.float32),
        pltpu.VMEM((STG,), jnp.int32),
        pltpu.VMEM((STG,), jnp.int32),
        pltpu.VMEM((BLK,), jnp.int32),
        pltpu.VMEM((BLK,), jnp.int32),
        pltpu.VMEM((BLK, D), jnp.float32),
        pltpu.VMEM((BLK,), jnp.int32),
        pltpu.VMEM((BLK,), jnp.int32),
        pltpu.VMEM((BLK, D), jnp.float32),
        pltpu.SemaphoreType.DMA,
        pltpu.SemaphoreType.DMA,
        pltpu.SemaphoreType.DMA,
        pltpu.SemaphoreType.DMA,
    ],
)
def _segsum(table, srcp, dstp, zeros_h, out,
            acc, sidx_c, didx_c, lblk, sblk, rows, lblk2, sblk2, rows2,
            sem, sem2, sem3, sem4):
    _segsum_body(table, srcp, dstp, zeros_h, out,
                 acc, sidx_c, didx_c, lblk, sblk, rows, lblk2, sblk2, rows2,
                 sem, sem2, sem3, sem4)


def _counts_body(dsts, ones_h, zeros_h, outs, cnt, didx_c, lblk, lblk2,
                 onesv, sem, sem2):
    c = lax.axis_index("c")
    s = lax.axis_index("s")
    pltpu.sync_copy(ones_h, onesv)
    for e in range(3):
        for r in range(ROUNDS):
            lo = (2 * r + c) * CHUNK
            base = s * (ACCR // 16)
            for qo, qn in _ZSLICES:
                pltpu.sync_copy(zeros_h.at[pl.ds(0, qn)],
                                cnt.at[pl.ds(base + qo, qn)])
            plsc.subcore_barrier()
            for soff, ssz in _stage_sizes():
                pltpu.sync_copy(dsts[e].at[pl.ds(s * EPT + soff, ssz)],
                                didx_c.at[pl.ds(0, ssz)])

                def epair(i, carry):
                    o0 = 2 * i * BLK
                    for k in range(BLK // 16):
                        d = didx_c[pl.ds(o0 + k * 16, 16)]
                        inr = (d >= lo) & (d < lo + CHUNK)
                        lblk[pl.ds(k * 16, 16)] = jnp.where(inr, d - lo,
                                                            CHUNK)
                    s0 = pltpu.async_copy(onesv, cnt.at[lblk], sem, add=True)
                    o1 = o0 + BLK
                    for k in range(BLK // 16):
                        d = didx_c[pl.ds(o1 + k * 16, 16)]
                        inr = (d >= lo) & (d < lo + CHUNK)
                        lblk2[pl.ds(k * 16, 16)] = jnp.where(inr, d - lo,
                                                             CHUNK)
                    s1 = pltpu.async_copy(onesv, cnt.at[lblk2], sem2,
                                          add=True)
                    s0.wait()
                    s1.wait()
                    return carry

                lax.fori_loop(0, ssz // (2 * BLK), epair, 0)
            plsc.subcore_barrier()
            wbase = s * (CHUNK // 16)
            for qo, qn in _WSLICES:
                pltpu.sync_copy(cnt.at[pl.ds(wbase + qo, qn)],
                                outs[e].at[pl.ds(lo + wbase + qo, qn)])
            plsc.subcore_barrier()


@functools.partial(
    pl.kernel,
    out_type=[jax.ShapeDtypeStruct((OUTPAD, D), jnp.float32)] * 3,
    mesh=_MESH,
    scratch_types=[
        pltpu.VMEM_SHARED((ACCR, D), jnp.float32),
        pltpu.VMEM((STG,), jnp.int32),
        pltpu.VMEM((BLK,), jnp.int32),
        pltpu.VMEM((BLK,), jnp.int32),
        pltpu.VMEM((BLK, D), jnp.float32),
        pltpu.SemaphoreType.DMA,
        pltpu.SemaphoreType.DMA,
    ],
)
def _counts(dst_f, dst_c, dst_cb, ones_h, zeros_h, out_f, out_c, out_cb,
            cnt, didx_c, lblk, lblk2, onesv, sem, sem2):
    _counts_body((dst_f, dst_c, dst_cb), ones_h, zeros_h,
                 (out_f, out_c, out_cb), cnt, didx_c, lblk, lblk2, onesv,
                 sem, sem2)


RBLK = 400


def _combine_u_kernel(sf, cf, scb, ccb, fu, wf, wcb, w0, bf, bcb, b0, out,
                      *, leaky):
    cfv = cf[...]
    ccbv = ccb[...]
    xf = sf[...] / jnp.maximum(cfv, 1.0)
    xcb = scb[...] / jnp.maximum(ccbv, 1.0)
    acc = jnp.dot(xf, wf[...], preferred_element_type=jnp.float32)
    acc = acc + jnp.dot(xcb, wcb[...], preferred_element_type=jnp.float32)
    acc = acc + jnp.dot(fu[...], w0[...], preferred_element_type=jnp.float32)
    bias = (b0[...] + jnp.where(cfv > 0.5, bf[...], 0.0)
            + jnp.where(ccbv > 0.5, bcb[...], 0.0))
    y = acc + bias
    if leaky:
        y = jnp.where(y >= 0, y, 0.01 * y)
    out[...] = y


def _combine_i_kernel(sc_, cc, fi, wc, w0, bc, b0, out, *, leaky):
    ccv = cc[...]
    xc = sc_[...] / jnp.maximum(ccv, 1.0)
    acc = jnp.dot(xc, wc[...], preferred_element_type=jnp.float32)
    acc = acc + jnp.dot(fi[...], w0[...], preferred_element_type=jnp.float32)
    bias = b0[...] + jnp.where(ccv > 0.5, bc[...], 0.0)
    y = acc + bias
    if leaky:
        y = jnp.where(y >= 0, y, 0.01 * y)
    out[...] = y


def _row_spec():
    return pl.BlockSpec((RBLK, D), lambda i: (i, 0))


def _cnt_spec():
    return pl.BlockSpec((RBLK, 1), lambda i: (i, 0))


def _w_spec():
    return pl.BlockSpec((D, D), lambda i: (0, 0))


def _b_spec():
    return pl.BlockSpec((1, D), lambda i: (0, 0))


def _combine_u(sf, cf, scb, ccb, fu, wf, wcb, w0, bf, bcb, b0, leaky):
    return pl.pallas_call(
        functools.partial(_combine_u_kernel, leaky=leaky),
        grid=(NU // RBLK,),
        in_specs=[_row_spec(), _cnt_spec(), _row_spec(), _cnt_spec(),
                  _row_spec(), _w_spec(), _w_spec(), _w_spec(),
                  _b_spec(), _b_spec(), _b_spec()],
        out_specs=_row_spec(),
        out_shape=jax.ShapeDtypeStruct((NU, D), jnp.float32),
    )(sf, cf, scb, ccb, fu, wf, wcb, w0,
      bf.reshape(1, D), bcb.reshape(1, D), b0.reshape(1, D))


def _combine_i(sc_, cc, fi, wc, w0, bc, b0, leaky):
    return pl.pallas_call(
        functools.partial(_combine_i_kernel, leaky=leaky),
        grid=(NI // RBLK,),
        in_specs=[_row_spec(), _cnt_spec(), _row_spec(),
                  _w_spec(), _w_spec(), _b_spec(), _b_spec()],
        out_specs=_row_spec(),
        out_shape=jax.ShapeDtypeStruct((NI, D), jnp.float32),
    )(sc_, cc, fi, wc, w0, bc.reshape(1, D), b0.reshape(1, D))


def _pad_edges(e):
    src = jnp.concatenate([e[0], jnp.zeros((EPAD - NE,), jnp.int32)])
    dst = jnp.concatenate([e[1], jnp.full((EPAD - NE,), NU, jnp.int32)])
    return src, dst


def kernel(e_follows, e_clicks, e_clickedby, emb_user, emb_item,
           l1_w0, l1_b0, l1_wf, l1_bf, l1_wc, l1_bc, l1_wcb, l1_bcb,
           l2_w0, l2_b0, l2_wf, l2_bf, l2_wc, l2_bc, l2_wcb, l2_bcb):
    src_f, dst_f = _pad_edges(e_follows)
    src_c, dst_c = _pad_edges(e_clicks)
    src_cb, dst_cb = _pad_edges(e_clickedby)
    zeros_h = jnp.zeros((128, D), jnp.float32)
    ones_hD = jnp.ones((BLK, D), jnp.float32)

    cnt_f, cnt_c, cnt_cb = _counts(dst_f, dst_c, dst_cb, ones_hD, zeros_h)
    cf = cnt_f[:NU, 0:1]
    cc = cnt_c[:NI, 0:1]
    ccb = cnt_cb[:NU, 0:1]

    def seg(tab, src, dst):
        return _segsum(tab, src, dst, zeros_h)[:NU]

    def layer(fu, fi, w0, b0, wf, bf, wc, bc, wcb, bcb, leaky):
        sf = seg(fu, src_f, dst_f)
        scb = seg(fi, src_cb, dst_cb)
        sc_ = seg(fu, src_c, dst_c)
        hu = _combine_u(sf, cf, scb, ccb, fu, wf, wcb, w0, bf, bcb, b0,
                        leaky)
        hi = _combine_i(sc_, cc, fi, wc, w0, bc, b0, leaky)
        return hu, hi

    hu, hi = layer(emb_user, emb_item, l1_w0, l1_b0, l1_wf, l1_bf,
                   l1_wc, l1_bc, l1_wcb, l1_bcb, True)
    hu, hi = layer(hu, hi, l2_w0, l2_b0, l2_wf, l2_bf,
                   l2_wc, l2_bc, l2_wcb, l2_bcb, False)
    return hu, hi
